# Initial kernel scaffold; baseline (speedup 1.0000x reference)
import jax
import jax.numpy as jnp
from jax.experimental import pallas as pl


def kernel(x, edge_index, edge_attr, conv_Wrel, conv_brel, conv_Wroot, lin_W, lin_b):
    raise NotImplementedError("write your pallas kernel here")


# speedup vs baseline: 7.2222x; 7.2222x over previous
import functools

import jax
import jax.numpy as jnp
from jax import lax
from jax.experimental import pallas as pl
from jax.experimental.pallas import tpu as pltpu
from jax.experimental.pallas import tpu_sc as plsc

N_REAL = 99990
N_PAD = 102400
CP = 16
NTILES = 32
CHUNK = 128
CHUNKS_PER_TILE = 1563
EDGES_PER_TILE = CHUNK * CHUNKS_PER_TILE
E_PAD = EDGES_PER_TILE * NTILES
ROWS_PER_SUB = N_PAD // 16
ZROWS = 256

TC_BLK = 2048
TC_GRID = N_PAD // TC_BLK


def _sc_edge_body(g_hbm, src_hbm, dst_hbm, ew_hbm, out_hbm,
                  agg_sh, zbuf, src_v, dst_v, ew_v, rows_v, sem):
    c = lax.axis_index("c")
    s = lax.axis_index("s")
    tid = s * 2 + c

    def _zrow(i, carry):
        zbuf[i, :] = jnp.zeros((CP,), jnp.float32)
        return carry
    lax.fori_loop(0, ZROWS, _zrow, 0)

    def _zcopy(j, carry):
        pltpu.sync_copy(zbuf, agg_sh.at[pl.ds(s * ROWS_PER_SUB + j * ZROWS, ZROWS)])
        return carry
    lax.fori_loop(0, ROWS_PER_SUB // ZROWS, _zcopy, 0)
    plsc.subcore_barrier()

    base = tid * EDGES_PER_TILE

    def _chunk(i, carry):
        off = base + i * CHUNK
        pltpu.sync_copy(src_hbm.at[pl.ds(off, CHUNK)], src_v)
        pltpu.sync_copy(dst_hbm.at[pl.ds(off, CHUNK)], dst_v)
        pltpu.sync_copy(ew_hbm.at[pl.ds(off, CHUNK)], ew_v)
        pltpu.async_copy(g_hbm.at[src_v], rows_v, sem).wait()
        dnums = lax.GatherDimensionNumbers(
            offset_dims=(), collapsed_slice_dims=(0,), start_index_map=(0,))
        for gi in range(CHUNK // 16):
            ew16 = ew_v[pl.ds(gi * 16, 16)]
            for k in range(16):
                bcast = lax.gather(
                    ew16, jnp.full((16, 1), k, jnp.int32), dnums, (1,),
                    mode=lax.GatherScatterMode.PROMISE_IN_BOUNDS)
                e = gi * 16 + k
                rows_v[e, :] = rows_v[e, :] * bcast
        pltpu.sync_copy(rows_v, agg_sh.at[dst_v], add=True)
        return carry
    lax.fori_loop(0, CHUNKS_PER_TILE, _chunk, 0)

    plsc.subcore_barrier()
    pltpu.sync_copy(agg_sh.at[pl.ds(s * ROWS_PER_SUB, ROWS_PER_SUB)],
                    out_hbm.at[c, pl.ds(s * ROWS_PER_SUB, ROWS_PER_SUB)])


_SC_EDGE_CACHE = []


def _sc_edge(*args):
    if not _SC_EDGE_CACHE:
        _SC_EDGE_CACHE.append(functools.partial(
            pl.kernel,
            out_type=jax.ShapeDtypeStruct((2, N_PAD, CP), jnp.float32),
            mesh=plsc.VectorSubcoreMesh(core_axis_name="c",
                                        subcore_axis_name="s"),
            compiler_params=pltpu.CompilerParams(use_tc_tiling_on_sc=False),
            scratch_types=[
                pltpu.VMEM_SHARED((N_PAD, CP), jnp.float32),
                pltpu.VMEM((ZROWS, CP), jnp.float32),
                pltpu.VMEM((CHUNK,), jnp.int32),
                pltpu.VMEM((CHUNK,), jnp.int32),
                pltpu.VMEM((CHUNK,), jnp.float32),
                pltpu.VMEM((CHUNK, CP), jnp.float32),
                pltpu.SemaphoreType.DMA,
            ],
        )(_sc_edge_body))
    return _SC_EDGE_CACHE[0](*args)


def _dot(a, b):
    return lax.dot_general(a, b, (((1,), (0,)), ((), ())),
                           precision=lax.Precision.HIGHEST,
                           preferred_element_type=jnp.float32)


def _layer_body(first_layer, c_mid, aggp_ref, h_ref, wrel_t_ref, brel_ref,
                wroot_t_ref, wnext_t_ref, h_out_ref, g_out_ref):
    agg = aggp_ref[0] + aggp_ref[1]
    agg = agg[:, :c_mid]
    if first_layer:
        t = _dot(agg, wrel_t_ref[...])
    else:
        t = agg
    h = jnp.maximum(t + brel_ref[...] + _dot(h_ref[...], wroot_t_ref[...]), 0.0)
    h_out_ref[...] = h
    if wnext_t_ref is not None:
        g = _dot(h, wnext_t_ref[...])
        g_out_ref[...] = jnp.concatenate(
            [g, jnp.zeros((g.shape[0], CP - g.shape[1]), jnp.float32)], axis=1)


def _conv_layer_tc(first_layer, c_mid, c_in, c_out, c_next, aggp, h_prev,
                   wrel_t, brel, wroot_t, wnext_t):
    have_next = wnext_t is not None
    out_shape = [jax.ShapeDtypeStruct((N_PAD, c_out), jnp.float32)]
    out_specs = [pl.BlockSpec((TC_BLK, c_out), lambda i: (i, 0))]
    if have_next:
        out_shape.append(jax.ShapeDtypeStruct((N_PAD, CP), jnp.float32))
        out_specs.append(pl.BlockSpec((TC_BLK, CP), lambda i: (i, 0)))
    in_specs = [
        pl.BlockSpec((2, TC_BLK, CP), lambda i: (0, i, 0)),
        pl.BlockSpec((TC_BLK, c_in), lambda i: (i, 0)),
        pl.BlockSpec(wrel_t.shape, lambda i: (0, 0)),
        pl.BlockSpec(brel.shape, lambda i: (0,)),
        pl.BlockSpec(wroot_t.shape, lambda i: (0, 0)),
    ]
    args = [aggp, h_prev, wrel_t, brel, wroot_t]
    if have_next:
        in_specs.append(pl.BlockSpec(wnext_t.shape, lambda i: (0, 0)))
        args.append(wnext_t)
        body = functools.partial(_layer_body, first_layer, c_mid)
    else:
        def body(a, h, wr, br, wo, ho):
            _layer_body(first_layer, c_mid, a, h, wr, br, wo, None, ho, None)
    return pl.pallas_call(
        body,
        grid=(TC_GRID,),
        in_specs=in_specs,
        out_specs=out_specs if have_next else out_specs[0],
        out_shape=out_shape if have_next else out_shape[0],
    )(*args)


def _mlp_body(h_ref, *refs):
    out_ref = refs[-1]
    wb = refs[:-1]
    h = h_ref[...]
    n_lin = len(wb) // 2
    for li in range(n_lin):
        h = _dot(h, wb[2 * li][...]) + wb[2 * li + 1][...]
        if li < n_lin - 1:
            h = jnp.maximum(h, 0.0)
    m = jnp.max(h, axis=1, keepdims=True)
    e = jnp.exp(h - m)
    out_ref[...] = e / jnp.sum(e, axis=1, keepdims=True)


def _mlp_tc(h, lin_W, lin_b):
    args = [h]
    for W, b in zip(lin_W, lin_b):
        args.append(W.T)
        args.append(b)
    rows = h.shape[0]
    return pl.pallas_call(
        _mlp_body,
        out_shape=jax.ShapeDtypeStruct((rows, 2), jnp.float32),
    )(*args)


def kernel(x, edge_index, edge_attr, conv_Wrel, conv_brel, conv_Wroot,
           lin_W, lin_b):
    pad_e = E_PAD - edge_index.shape[1]
    spread = (jnp.arange(pad_e, dtype=jnp.int32) * 131) % jnp.int32(N_REAL)
    src = jnp.concatenate([edge_index[0].astype(jnp.int32), spread])
    dst = jnp.concatenate([edge_index[1].astype(jnp.int32), spread])
    ew = jnp.concatenate([edge_attr, jnp.zeros((pad_e,), jnp.float32)])

    x_pad = jnp.zeros((N_PAD, 2), jnp.float32).at[:N_REAL].set(x)
    g = jnp.zeros((N_PAD, CP), jnp.float32).at[:N_REAL, :2].set(x)

    h = x_pad
    n_layers = len(conv_Wrel)
    for l in range(n_layers):
        c_in = conv_Wrel[l].shape[1]
        c_out = conv_Wrel[l].shape[0]
        aggp = _sc_edge(g, src, dst, ew)
        first = (l == 0)
        c_mid = c_in if first else c_out
        wnext_t = conv_Wrel[l + 1].T if l + 1 < n_layers else None
        res = _conv_layer_tc(first, c_mid, c_in, c_out,
                             None if wnext_t is None else wnext_t.shape[1],
                             aggp, h, conv_Wrel[l].T, conv_brel[l],
                             conv_Wroot[l].T, wnext_t)
        if wnext_t is None:
            h = res
        else:
            h, g = res

    h5 = h[:N_REAL].reshape(-1, 396)
    rows = h5.shape[0]
    rows_pad = ((rows + 7) // 8) * 8
    h5p = jnp.zeros((rows_pad, 396), jnp.float32).at[:rows].set(h5)
    out = _mlp_tc(h5p, lin_W, lin_b)
    return out[:rows]


# speedup vs baseline: 20.9900x; 2.9063x over previous
import functools

import jax
import jax.numpy as jnp
from jax import lax
from jax.experimental import pallas as pl
from jax.experimental.pallas import tpu as pltpu
from jax.experimental.pallas import tpu_sc as plsc

N_REAL = 99990
N_PAD = 102400
CP = 16
NTILES = 32
CHUNK = 128
SUB = 8
NB = SUB * CHUNK
NBLOCKS = 196
EDGES_PER_TILE = NB * NBLOCKS
E_PAD = EDGES_PER_TILE * NTILES
NRING = 4
ROWS_PER_SUB = N_PAD // 16
ZROWS = 256

TC_BLK = 2048
TC_GRID = N_PAD // TC_BLK


def _sc_edge_body(g_hbm, src_hbm, dst2_hbm, ew_hbm, out_hbm,
                  agg_sh, zbuf, src_b, dst_b, ew_b, rows_b,
                  sem_stage, sem_g, sem_sc):
    c = lax.axis_index("c")
    s = lax.axis_index("s")
    tid = s * 2 + c

    def _zrow(i, carry):
        zbuf[i, :] = jnp.zeros((CP,), jnp.float32)
        return carry
    lax.fori_loop(0, ZROWS, _zrow, 0)

    def _zcopy(j, carry):
        pltpu.sync_copy(zbuf, agg_sh.at[pl.ds(s * ROWS_PER_SUB + j * ZROWS, ZROWS)])
        return carry
    lax.fori_loop(0, ROWS_PER_SUB // ZROWS, _zcopy, 0)
    plsc.subcore_barrier()

    ebase = tid * EDGES_PER_TILE
    rbase = tid * (EDGES_PER_TILE // CHUNK)
    dnums = lax.GatherDimensionNumbers(
        offset_dims=(), collapsed_slice_dims=(0,), start_index_map=(0,))

    def _stage(b, buf):
        off = ebase + b * NB
        pltpu.async_copy(src_hbm.at[pl.ds(off, NB)], src_b.at[buf], sem_stage)
        pltpu.async_copy(ew_hbm.at[pl.ds(off, NB)], ew_b.at[buf], sem_stage)
        pltpu.async_copy(dst2_hbm.at[pl.ds(rbase + b * SUB, SUB)],
                         dst_b.at[buf], sem_stage)

    _stage(0, 0)

    def _block(b, carry):
        cur = lax.rem(b, 2)
        pltpu.make_async_copy(src_hbm.at[pl.ds(0, NB)], src_b.at[cur],
                              sem_stage).wait()
        pltpu.make_async_copy(ew_hbm.at[pl.ds(0, NB)], ew_b.at[cur],
                              sem_stage).wait()
        pltpu.make_async_copy(dst2_hbm.at[pl.ds(0, SUB)], dst_b.at[cur],
                              sem_stage).wait()

        @pl.when(b + 1 < NBLOCKS)
        def _():
            _stage(b + 1, 1 - cur)

        def _gather(j, ring):
            return pltpu.async_copy(
                g_hbm.at[src_b.at[cur, pl.ds(j * CHUNK, CHUNK)]],
                rows_b.at[ring], sem_g)

        g_descs = [None] * SUB
        sc_descs = [None] * SUB
        g_descs[0] = _gather(0, 0)
        for j in range(SUB):
            if j + 1 < SUB:
                if j + 1 >= NRING:
                    sc_descs[j + 1 - NRING].wait()
                g_descs[j + 1] = _gather(j + 1, (j + 1) % NRING)
            g_descs[j].wait()
            r = j % NRING
            for gi in range(CHUNK // 16):
                ew16 = ew_b[cur, pl.ds(j * CHUNK + gi * 16, 16)]
                for k in range(16):
                    bcast = lax.gather(
                        ew16, jnp.full((16, 1), k, jnp.int32), dnums, (1,),
                        mode=lax.GatherScatterMode.PROMISE_IN_BOUNDS)
                    e = gi * 16 + k
                    rows_b[r, e, :] = rows_b[r, e, :] * bcast
            sc_descs[j] = pltpu.async_copy(
                rows_b.at[r], agg_sh.at[dst_b.at[cur, j]], sem_sc, add=True)
        for j in range(SUB - NRING, SUB):
            sc_descs[j].wait()
        return carry

    lax.fori_loop(0, NBLOCKS, _block, 0)

    plsc.subcore_barrier()
    pltpu.sync_copy(agg_sh.at[pl.ds(s * ROWS_PER_SUB, ROWS_PER_SUB)],
                    out_hbm.at[c, pl.ds(s * ROWS_PER_SUB, ROWS_PER_SUB)])


_SC_EDGE_CACHE = []


def _sc_edge(*args):
    if not _SC_EDGE_CACHE:
        _SC_EDGE_CACHE.append(functools.partial(
            pl.kernel,
            out_type=jax.ShapeDtypeStruct((2, N_PAD, CP), jnp.float32),
            mesh=plsc.VectorSubcoreMesh(core_axis_name="c",
                                        subcore_axis_name="s"),
            compiler_params=pltpu.CompilerParams(use_tc_tiling_on_sc=False),
            scratch_types=[
                pltpu.VMEM_SHARED((N_PAD, CP), jnp.float32),
                pltpu.VMEM((ZROWS, CP), jnp.float32),
                pltpu.VMEM((2, NB), jnp.int32),
                pltpu.VMEM((2, SUB, CHUNK), jnp.int32),
                pltpu.VMEM((2, NB), jnp.float32),
                pltpu.VMEM((NRING, CHUNK, CP), jnp.float32),
                pltpu.SemaphoreType.DMA,
                pltpu.SemaphoreType.DMA,
                pltpu.SemaphoreType.DMA,
            ],
        )(_sc_edge_body))
    return _SC_EDGE_CACHE[0](*args)


def _dot(a, b):
    return lax.dot_general(a, b, (((1,), (0,)), ((), ())),
                           precision=lax.Precision.HIGHEST,
                           preferred_element_type=jnp.float32)


def _layer_body(first_layer, c_mid, aggp_ref, h_ref, wrel_t_ref, brel_ref,
                wroot_t_ref, wnext_t_ref, h_out_ref, g_out_ref):
    agg = aggp_ref[0] + aggp_ref[1]
    agg = agg[:, :c_mid]
    if first_layer:
        t = _dot(agg, wrel_t_ref[...])
    else:
        t = agg
    h = jnp.maximum(t + brel_ref[...] + _dot(h_ref[...], wroot_t_ref[...]), 0.0)
    h_out_ref[...] = h
    if wnext_t_ref is not None:
        g = _dot(h, wnext_t_ref[...])
        g_out_ref[...] = jnp.concatenate(
            [g, jnp.zeros((g.shape[0], CP - g.shape[1]), jnp.float32)], axis=1)


def _conv_layer_tc(first_layer, c_mid, c_in, c_out, c_next, aggp, h_prev,
                   wrel_t, brel, wroot_t, wnext_t):
    have_next = wnext_t is not None
    out_shape = [jax.ShapeDtypeStruct((N_PAD, c_out), jnp.float32)]
    out_specs = [pl.BlockSpec((TC_BLK, c_out), lambda i: (i, 0))]
    if have_next:
        out_shape.append(jax.ShapeDtypeStruct((N_PAD, CP), jnp.float32))
        out_specs.append(pl.BlockSpec((TC_BLK, CP), lambda i: (i, 0)))
    in_specs = [
        pl.BlockSpec((2, TC_BLK, CP), lambda i: (0, i, 0)),
        pl.BlockSpec((TC_BLK, c_in), lambda i: (i, 0)),
        pl.BlockSpec(wrel_t.shape, lambda i: (0, 0)),
        pl.BlockSpec(brel.shape, lambda i: (0,)),
        pl.BlockSpec(wroot_t.shape, lambda i: (0, 0)),
    ]
    args = [aggp, h_prev, wrel_t, brel, wroot_t]
    if have_next:
        in_specs.append(pl.BlockSpec(wnext_t.shape, lambda i: (0, 0)))
        args.append(wnext_t)
        body = functools.partial(_layer_body, first_layer, c_mid)
    else:
        def body(a, h, wr, br, wo, ho):
            _layer_body(first_layer, c_mid, a, h, wr, br, wo, None, ho, None)
    return pl.pallas_call(
        body,
        grid=(TC_GRID,),
        in_specs=in_specs,
        out_specs=out_specs if have_next else out_specs[0],
        out_shape=out_shape if have_next else out_shape[0],
    )(*args)


def _mlp_body(h_ref, *refs):
    out_ref = refs[-1]
    wb = refs[:-1]
    h = h_ref[...]
    n_lin = len(wb) // 2
    for li in range(n_lin):
        h = _dot(h, wb[2 * li][...]) + wb[2 * li + 1][...]
        if li < n_lin - 1:
            h = jnp.maximum(h, 0.0)
    m = jnp.max(h, axis=1, keepdims=True)
    e = jnp.exp(h - m)
    out_ref[...] = e / jnp.sum(e, axis=1, keepdims=True)


def _mlp_tc(h, lin_W, lin_b):
    args = [h]
    for W, b in zip(lin_W, lin_b):
        args.append(W.T)
        args.append(b)
    rows = h.shape[0]
    return pl.pallas_call(
        _mlp_body,
        out_shape=jax.ShapeDtypeStruct((rows, 2), jnp.float32),
    )(*args)


def kernel(x, edge_index, edge_attr, conv_Wrel, conv_brel, conv_Wroot,
           lin_W, lin_b):
    pad_e = E_PAD - edge_index.shape[1]
    spread = (jnp.arange(pad_e, dtype=jnp.int32) * 131) % jnp.int32(N_REAL)
    src = jnp.concatenate([edge_index[0].astype(jnp.int32), spread])
    dst = jnp.concatenate([edge_index[1].astype(jnp.int32), spread])
    dst2 = dst.reshape(E_PAD // CHUNK, CHUNK)
    ew = jnp.concatenate([edge_attr, jnp.zeros((pad_e,), jnp.float32)])

    x_pad = jnp.zeros((N_PAD, 2), jnp.float32).at[:N_REAL].set(x)
    g = jnp.zeros((N_PAD, CP), jnp.float32).at[:N_REAL, :2].set(x)

    h = x_pad
    n_layers = len(conv_Wrel)
    for l in range(n_layers):
        c_in = conv_Wrel[l].shape[1]
        c_out = conv_Wrel[l].shape[0]
        aggp = _sc_edge(g, src, dst2, ew)
        first = (l == 0)
        c_mid = c_in if first else c_out
        wnext_t = conv_Wrel[l + 1].T if l + 1 < n_layers else None
        res = _conv_layer_tc(first, c_mid, c_in, c_out,
                             None if wnext_t is None else wnext_t.shape[1],
                             aggp, h, conv_Wrel[l].T, conv_brel[l],
                             conv_Wroot[l].T, wnext_t)
        if wnext_t is None:
            h = res
        else:
            h, g = res

    h5 = h[:N_REAL].reshape(-1, 396)
    rows = h5.shape[0]
    rows_pad = ((rows + 7) // 8) * 8
    h5p = jnp.zeros((rows_pad, 396), jnp.float32).at[:rows].set(h5)
    out = _mlp_tc(h5p, lin_W, lin_b)
    return out[:rows]


# speedup vs baseline: 26.2218x; 1.2493x over previous
import functools

import jax
import jax.numpy as jnp
from jax import lax
from jax.experimental import pallas as pl
from jax.experimental.pallas import tpu as pltpu
from jax.experimental.pallas import tpu_sc as plsc

N_REAL = 99990
N_PAD = 100352
CP = 16
NTILES = 32
CHUNK = 128
SUB = 8
NB = SUB * CHUNK
NBLOCKS = 196
EDGES_PER_TILE = NB * NBLOCKS
E_PAD = EDGES_PER_TILE * NTILES
NRING = 8
AHEAD = 3
ROWS_PER_SUB = N_PAD // 16
ZROWS = 128

TC_BLK = 2048
TC_GRID = N_PAD // TC_BLK


def _sc_edge_body(g_hbm, src2_hbm, dst2_hbm, ew2_hbm, out_hbm,
                  agg_sh, zbuf, src_b, dst_b, ew_b, rows_b,
                  sem_stage, *sems):
    sem_g = sems[:NRING]
    sem_s = sems[NRING:]
    c = lax.axis_index("c")
    s = lax.axis_index("s")
    tid = s * 2 + c

    def _zrow(i, carry):
        zbuf[i, :] = jnp.zeros((CP,), jnp.float32)
        return carry
    lax.fori_loop(0, ZROWS, _zrow, 0)

    def _zcopy(j, carry):
        pltpu.sync_copy(zbuf, agg_sh.at[pl.ds(s * ROWS_PER_SUB + j * ZROWS, ZROWS)])
        return carry
    lax.fori_loop(0, ROWS_PER_SUB // ZROWS, _zcopy, 0)
    plsc.subcore_barrier()

    rbase = tid * (EDGES_PER_TILE // CHUNK)
    dnums = lax.GatherDimensionNumbers(
        offset_dims=(), collapsed_slice_dims=(0,), start_index_map=(0,))

    def _stage(b):
        slot = lax.rem(b, 3)
        row0 = rbase + b * SUB
        pltpu.async_copy(src2_hbm.at[pl.ds(row0, SUB)], src_b.at[slot],
                         sem_stage)
        pltpu.async_copy(ew2_hbm.at[pl.ds(row0, SUB)], ew_b.at[slot],
                         sem_stage)
        pltpu.async_copy(dst2_hbm.at[pl.ds(row0, SUB)], dst_b.at[slot],
                         sem_stage)

    def _wait_stage():
        pltpu.make_async_copy(src2_hbm.at[pl.ds(0, SUB)], src_b.at[0],
                              sem_stage).wait()
        pltpu.make_async_copy(ew2_hbm.at[pl.ds(0, SUB)], ew_b.at[0],
                              sem_stage).wait()
        pltpu.make_async_copy(dst2_hbm.at[pl.ds(0, SUB)], dst_b.at[0],
                              sem_stage).wait()

    def _fire_gather(bslot, j):
        pltpu.async_copy(g_hbm.at[src_b.at[bslot, j]], rows_b.at[j],
                         sem_g[j])

    def _wait_gather(j):
        pltpu.make_async_copy(g_hbm.at[pl.ds(0, CHUNK)], rows_b.at[j],
                              sem_g[j]).wait()

    def _fire_scatter(bslot, j):
        pltpu.async_copy(rows_b.at[j], agg_sh.at[dst_b.at[bslot, j]],
                         sem_s[j], add=True)

    def _wait_scatter(j):
        pltpu.make_async_copy(rows_b.at[j], agg_sh.at[dst_b.at[0, j]],
                              sem_s[j]).wait()

    _stage(0)
    _wait_stage()
    _stage(1)
    for j in range(AHEAD):
        _fire_gather(0, j)

    def _block(b, carry):
        bslot = lax.rem(b, 3)
        nslot = lax.rem(b + 1, 3)
        for j in range(SUB):
            if j + AHEAD < SUB:
                @pl.when(b > 0)
                def _():
                    _wait_scatter(j + AHEAD)
                _fire_gather(bslot, j + AHEAD)
            _wait_gather(j)
            for gi in range(CHUNK // 16):
                ew16 = ew_b[bslot, j, pl.ds(gi * 16, 16)]
                for k in range(16):
                    bcast = lax.gather(
                        ew16, jnp.full((16, 1), k, jnp.int32), dnums, (1,),
                        mode=lax.GatherScatterMode.PROMISE_IN_BOUNDS)
                    e = gi * 16 + k
                    rows_b[j, e, :] = rows_b[j, e, :] * bcast
            _fire_scatter(bslot, j)

        @pl.when(b + 1 < NBLOCKS)
        def _():
            _wait_stage()
            @pl.when(b + 2 < NBLOCKS)
            def _():
                _stage(b + 2)
            for j in range(AHEAD):
                _wait_scatter(j)
                _fire_gather(nslot, j)
        return carry

    lax.fori_loop(0, NBLOCKS, _block, 0)
    for j in range(AHEAD, SUB):
        _wait_scatter(j)
    for j in range(AHEAD):
        _wait_scatter(j)

    plsc.subcore_barrier()
    pltpu.sync_copy(agg_sh.at[pl.ds(s * ROWS_PER_SUB, ROWS_PER_SUB)],
                    out_hbm.at[c, pl.ds(s * ROWS_PER_SUB, ROWS_PER_SUB)])


_SC_EDGE_CACHE = []


def _sc_edge(*args):
    if not _SC_EDGE_CACHE:
        _SC_EDGE_CACHE.append(functools.partial(
            pl.kernel,
            out_type=jax.ShapeDtypeStruct((2, N_PAD, CP), jnp.float32),
            mesh=plsc.VectorSubcoreMesh(core_axis_name="c",
                                        subcore_axis_name="s"),
            compiler_params=pltpu.CompilerParams(use_tc_tiling_on_sc=False),
            scratch_types=[
                pltpu.VMEM_SHARED((N_PAD, CP), jnp.float32),
                pltpu.VMEM((ZROWS, CP), jnp.float32),
                pltpu.VMEM((3, SUB, CHUNK), jnp.int32),
                pltpu.VMEM((3, SUB, CHUNK), jnp.int32),
                pltpu.VMEM((3, SUB, CHUNK), jnp.float32),
                pltpu.VMEM((NRING, CHUNK, CP), jnp.float32),
                pltpu.SemaphoreType.DMA,
            ] + [pltpu.SemaphoreType.DMA] * (2 * NRING),
        )(_sc_edge_body))
    return _SC_EDGE_CACHE[0](*args)


def _dot(a, b):
    return lax.dot_general(a, b, (((1,), (0,)), ((), ())),
                           precision=lax.Precision.HIGHEST,
                           preferred_element_type=jnp.float32)


def _layer_body(first_layer, c_mid, aggp_ref, h_ref, wrel_t_ref, brel_ref,
                wroot_t_ref, wnext_t_ref, h_out_ref, g_out_ref):
    agg = aggp_ref[0] + aggp_ref[1]
    agg = agg[:, :c_mid]
    if first_layer:
        t = _dot(agg, wrel_t_ref[...])
    else:
        t = agg
    h = jnp.maximum(t + brel_ref[...] + _dot(h_ref[...], wroot_t_ref[...]), 0.0)
    h_out_ref[...] = h
    if wnext_t_ref is not None:
        g = _dot(h, wnext_t_ref[...])
        g_out_ref[...] = jnp.concatenate(
            [g, jnp.zeros((g.shape[0], CP - g.shape[1]), jnp.float32)], axis=1)


def _conv_layer_tc(first_layer, c_mid, c_in, c_out, c_next, aggp, h_prev,
                   wrel_t, brel, wroot_t, wnext_t):
    have_next = wnext_t is not None
    out_shape = [jax.ShapeDtypeStruct((N_PAD, c_out), jnp.float32)]
    out_specs = [pl.BlockSpec((TC_BLK, c_out), lambda i: (i, 0))]
    if have_next:
        out_shape.append(jax.ShapeDtypeStruct((N_PAD, CP), jnp.float32))
        out_specs.append(pl.BlockSpec((TC_BLK, CP), lambda i: (i, 0)))
    in_specs = [
        pl.BlockSpec((2, TC_BLK, CP), lambda i: (0, i, 0)),
        pl.BlockSpec((TC_BLK, c_in), lambda i: (i, 0)),
        pl.BlockSpec(wrel_t.shape, lambda i: (0, 0)),
        pl.BlockSpec(brel.shape, lambda i: (0,)),
        pl.BlockSpec(wroot_t.shape, lambda i: (0, 0)),
    ]
    args = [aggp, h_prev, wrel_t, brel, wroot_t]
    if have_next:
        in_specs.append(pl.BlockSpec(wnext_t.shape, lambda i: (0, 0)))
        args.append(wnext_t)
        body = functools.partial(_layer_body, first_layer, c_mid)
    else:
        def body(a, h, wr, br, wo, ho):
            _layer_body(first_layer, c_mid, a, h, wr, br, wo, None, ho, None)
    return pl.pallas_call(
        body,
        grid=(TC_GRID,),
        in_specs=in_specs,
        out_specs=out_specs if have_next else out_specs[0],
        out_shape=out_shape if have_next else out_shape[0],
    )(*args)


def _mlp_body(h_ref, *refs):
    out_ref = refs[-1]
    wb = refs[:-1]
    h = h_ref[...]
    n_lin = len(wb) // 2
    for li in range(n_lin):
        h = _dot(h, wb[2 * li][...]) + wb[2 * li + 1][...]
        if li < n_lin - 1:
            h = jnp.maximum(h, 0.0)
    m = jnp.max(h, axis=1, keepdims=True)
    e = jnp.exp(h - m)
    out_ref[...] = e / jnp.sum(e, axis=1, keepdims=True)


def _mlp_tc(h, lin_W, lin_b):
    args = [h]
    for W, b in zip(lin_W, lin_b):
        args.append(W.T)
        args.append(b)
    rows = h.shape[0]
    return pl.pallas_call(
        _mlp_body,
        out_shape=jax.ShapeDtypeStruct((rows, 2), jnp.float32),
    )(*args)


def kernel(x, edge_index, edge_attr, conv_Wrel, conv_brel, conv_Wroot,
           lin_W, lin_b):
    pad_e = E_PAD - edge_index.shape[1]
    spread = (jnp.arange(pad_e, dtype=jnp.int32) * 131) % jnp.int32(N_REAL)
    src = jnp.concatenate([edge_index[0].astype(jnp.int32), spread])
    dst = jnp.concatenate([edge_index[1].astype(jnp.int32), spread])
    src2 = src.reshape(E_PAD // CHUNK, CHUNK)
    dst2 = dst.reshape(E_PAD // CHUNK, CHUNK)
    ew = jnp.concatenate([edge_attr, jnp.zeros((pad_e,), jnp.float32)])
    ew2 = ew.reshape(E_PAD // CHUNK, CHUNK)

    x_pad = jnp.zeros((N_PAD, 2), jnp.float32).at[:N_REAL].set(x)
    g = jnp.zeros((N_PAD, CP), jnp.float32).at[:N_REAL, :2].set(x)

    h = x_pad
    n_layers = len(conv_Wrel)
    for l in range(n_layers):
        c_in = conv_Wrel[l].shape[1]
        c_out = conv_Wrel[l].shape[0]
        aggp = _sc_edge(g, src2, dst2, ew2)
        first = (l == 0)
        c_mid = c_in if first else c_out
        wnext_t = conv_Wrel[l + 1].T if l + 1 < n_layers else None
        res = _conv_layer_tc(first, c_mid, c_in, c_out,
                             None if wnext_t is None else wnext_t.shape[1],
                             aggp, h, conv_Wrel[l].T, conv_brel[l],
                             conv_Wroot[l].T, wnext_t)
        if wnext_t is None:
            h = res
        else:
            h, g = res

    h5 = h[:N_REAL].reshape(-1, 396)
    rows = h5.shape[0]
    rows_pad = ((rows + 7) // 8) * 8
    h5p = jnp.zeros((rows_pad, 396), jnp.float32).at[:rows].set(h5)
    out = _mlp_tc(h5p, lin_W, lin_b)
    return out[:rows]


# speedup vs baseline: 26.5851x; 1.0139x over previous
import functools

import jax
import jax.numpy as jnp
from jax import lax
from jax.experimental import pallas as pl
from jax.experimental.pallas import tpu as pltpu
from jax.experimental.pallas import tpu_sc as plsc

N_REAL = 99990
N_PAD = 100352
CP = 16
NTILES = 32
CHUNK = 128
SUB = 8
NB = SUB * CHUNK
NBLOCKS = 196
EDGES_PER_TILE = NB * NBLOCKS
E_PAD = EDGES_PER_TILE * NTILES
NRING = 8
AHEAD = 4
ROWS_PER_SUB = N_PAD // 16
ZROWS = 128

TC_BLK = 2048
TC_GRID = N_PAD // TC_BLK


def _sc_edge_body(g_hbm, src2_hbm, dst2_hbm, ew2_hbm, out_hbm,
                  agg_sh, zbuf, src_b, dst_b, ew_b, rows_b,
                  sem_stage, *sems):
    sem_g = sems[:NRING]
    sem_s = sems[NRING:]
    c = lax.axis_index("c")
    s = lax.axis_index("s")
    tid = s * 2 + c

    def _zrow(i, carry):
        zbuf[i, :] = jnp.zeros((CP,), jnp.float32)
        return carry
    lax.fori_loop(0, ZROWS, _zrow, 0)

    def _zcopy(j, carry):
        pltpu.sync_copy(zbuf, agg_sh.at[pl.ds(s * ROWS_PER_SUB + j * ZROWS, ZROWS)])
        return carry
    lax.fori_loop(0, ROWS_PER_SUB // ZROWS, _zcopy, 0)
    plsc.subcore_barrier()

    rbase = tid * (EDGES_PER_TILE // CHUNK)
    dnums = lax.GatherDimensionNumbers(
        offset_dims=(), collapsed_slice_dims=(0,), start_index_map=(0,))

    def _stage(b):
        slot = lax.rem(b, 3)
        row0 = rbase + b * SUB
        pltpu.async_copy(src2_hbm.at[pl.ds(row0, SUB)], src_b.at[slot],
                         sem_stage)
        pltpu.async_copy(ew2_hbm.at[pl.ds(row0, SUB)], ew_b.at[slot],
                         sem_stage)
        pltpu.async_copy(dst2_hbm.at[pl.ds(row0, SUB)], dst_b.at[slot],
                         sem_stage)

    def _wait_stage():
        pltpu.make_async_copy(src2_hbm.at[pl.ds(0, SUB)], src_b.at[0],
                              sem_stage).wait()
        pltpu.make_async_copy(ew2_hbm.at[pl.ds(0, SUB)], ew_b.at[0],
                              sem_stage).wait()
        pltpu.make_async_copy(dst2_hbm.at[pl.ds(0, SUB)], dst_b.at[0],
                              sem_stage).wait()

    def _fire_gather(bslot, j):
        pltpu.async_copy(g_hbm.at[src_b.at[bslot, j]], rows_b.at[j],
                         sem_g[j])

    def _wait_gather(j):
        pltpu.make_async_copy(g_hbm.at[pl.ds(0, CHUNK)], rows_b.at[j],
                              sem_g[j]).wait()

    def _fire_scatter(bslot, j):
        pltpu.async_copy(rows_b.at[j], agg_sh.at[dst_b.at[bslot, j]],
                         sem_s[j], add=True)

    def _wait_scatter(j):
        pltpu.make_async_copy(rows_b.at[j], agg_sh.at[dst_b.at[0, j]],
                              sem_s[j]).wait()

    _stage(0)
    _wait_stage()
    _stage(1)
    for j in range(AHEAD):
        _fire_gather(0, j)

    def _block(b, carry):
        bslot = lax.rem(b, 3)
        nslot = lax.rem(b + 1, 3)
        for j in range(SUB):
            if j + AHEAD < SUB:
                @pl.when(b > 0)
                def _():
                    _wait_scatter(j + AHEAD)
                _fire_gather(bslot, j + AHEAD)
            _wait_gather(j)
            for gi in range(CHUNK // 16):
                ew16 = ew_b[bslot, j, pl.ds(gi * 16, 16)]
                for k in range(16):
                    bcast = lax.gather(
                        ew16, jnp.full((16, 1), k, jnp.int32), dnums, (1,),
                        mode=lax.GatherScatterMode.PROMISE_IN_BOUNDS)
                    e = gi * 16 + k
                    rows_b[j, e, :] = rows_b[j, e, :] * bcast
            _fire_scatter(bslot, j)

        @pl.when(b + 1 < NBLOCKS)
        def _():
            _wait_stage()
            @pl.when(b + 2 < NBLOCKS)
            def _():
                _stage(b + 2)
            for j in range(AHEAD):
                _wait_scatter(j)
                _fire_gather(nslot, j)
        return carry

    lax.fori_loop(0, NBLOCKS, _block, 0)
    for j in range(AHEAD, SUB):
        _wait_scatter(j)
    for j in range(AHEAD):
        _wait_scatter(j)

    plsc.subcore_barrier()
    pltpu.sync_copy(agg_sh.at[pl.ds(s * ROWS_PER_SUB, ROWS_PER_SUB)],
                    out_hbm.at[c, pl.ds(s * ROWS_PER_SUB, ROWS_PER_SUB)])


_SC_EDGE_CACHE = []


def _sc_edge(*args):
    if not _SC_EDGE_CACHE:
        _SC_EDGE_CACHE.append(functools.partial(
            pl.kernel,
            out_type=jax.ShapeDtypeStruct((2, N_PAD, CP), jnp.float32),
            mesh=plsc.VectorSubcoreMesh(core_axis_name="c",
                                        subcore_axis_name="s"),
            compiler_params=pltpu.CompilerParams(use_tc_tiling_on_sc=False),
            scratch_types=[
                pltpu.VMEM_SHARED((N_PAD, CP), jnp.float32),
                pltpu.VMEM((ZROWS, CP), jnp.float32),
                pltpu.VMEM((3, SUB, CHUNK), jnp.int32),
                pltpu.VMEM((3, SUB, CHUNK), jnp.int32),
                pltpu.VMEM((3, SUB, CHUNK), jnp.float32),
                pltpu.VMEM((NRING, CHUNK, CP), jnp.float32),
                pltpu.SemaphoreType.DMA,
            ] + [pltpu.SemaphoreType.DMA] * (2 * NRING),
        )(_sc_edge_body))
    return _SC_EDGE_CACHE[0](*args)


def _dot(a, b):
    return lax.dot_general(a, b, (((1,), (0,)), ((), ())),
                           precision=lax.Precision.HIGHEST,
                           preferred_element_type=jnp.float32)


def _layer_body(first_layer, c_mid, aggp_ref, h_ref, wrel_t_ref, brel_ref,
                wroot_t_ref, wnext_t_ref, h_out_ref, g_out_ref):
    agg = aggp_ref[0] + aggp_ref[1]
    agg = agg[:, :c_mid]
    if first_layer:
        t = _dot(agg, wrel_t_ref[...])
    else:
        t = agg
    h = jnp.maximum(t + brel_ref[...] + _dot(h_ref[...], wroot_t_ref[...]), 0.0)
    h_out_ref[...] = h
    if wnext_t_ref is not None:
        g = _dot(h, wnext_t_ref[...])
        g_out_ref[...] = jnp.concatenate(
            [g, jnp.zeros((g.shape[0], CP - g.shape[1]), jnp.float32)], axis=1)


def _conv_layer_tc(first_layer, c_mid, c_in, c_out, c_next, aggp, h_prev,
                   wrel_t, brel, wroot_t, wnext_t):
    have_next = wnext_t is not None
    out_shape = [jax.ShapeDtypeStruct((N_PAD, c_out), jnp.float32)]
    out_specs = [pl.BlockSpec((TC_BLK, c_out), lambda i: (i, 0))]
    if have_next:
        out_shape.append(jax.ShapeDtypeStruct((N_PAD, CP), jnp.float32))
        out_specs.append(pl.BlockSpec((TC_BLK, CP), lambda i: (i, 0)))
    in_specs = [
        pl.BlockSpec((2, TC_BLK, CP), lambda i: (0, i, 0)),
        pl.BlockSpec((TC_BLK, c_in), lambda i: (i, 0)),
        pl.BlockSpec(wrel_t.shape, lambda i: (0, 0)),
        pl.BlockSpec(brel.shape, lambda i: (0,)),
        pl.BlockSpec(wroot_t.shape, lambda i: (0, 0)),
    ]
    args = [aggp, h_prev, wrel_t, brel, wroot_t]
    if have_next:
        in_specs.append(pl.BlockSpec(wnext_t.shape, lambda i: (0, 0)))
        args.append(wnext_t)
        body = functools.partial(_layer_body, first_layer, c_mid)
    else:
        def body(a, h, wr, br, wo, ho):
            _layer_body(first_layer, c_mid, a, h, wr, br, wo, None, ho, None)
    return pl.pallas_call(
        body,
        grid=(TC_GRID,),
        in_specs=in_specs,
        out_specs=out_specs if have_next else out_specs[0],
        out_shape=out_shape if have_next else out_shape[0],
    )(*args)


def _mlp_body(h_ref, *refs):
    out_ref = refs[-1]
    wb = refs[:-1]
    h = h_ref[...]
    n_lin = len(wb) // 2
    for li in range(n_lin):
        h = _dot(h, wb[2 * li][...]) + wb[2 * li + 1][...]
        if li < n_lin - 1:
            h = jnp.maximum(h, 0.0)
    m = jnp.max(h, axis=1, keepdims=True)
    e = jnp.exp(h - m)
    out_ref[...] = e / jnp.sum(e, axis=1, keepdims=True)


def _mlp_tc(h, lin_W, lin_b):
    args = [h]
    for W, b in zip(lin_W, lin_b):
        args.append(W.T)
        args.append(b)
    rows = h.shape[0]
    return pl.pallas_call(
        _mlp_body,
        out_shape=jax.ShapeDtypeStruct((rows, 2), jnp.float32),
    )(*args)


def kernel(x, edge_index, edge_attr, conv_Wrel, conv_brel, conv_Wroot,
           lin_W, lin_b):
    pad_e = E_PAD - edge_index.shape[1]
    spread = (jnp.arange(pad_e, dtype=jnp.int32) * 131) % jnp.int32(N_REAL)
    src = jnp.concatenate([edge_index[0].astype(jnp.int32), spread])
    dst = jnp.concatenate([edge_index[1].astype(jnp.int32), spread])
    src2 = src.reshape(E_PAD // CHUNK, CHUNK)
    dst2 = dst.reshape(E_PAD // CHUNK, CHUNK)
    ew = jnp.concatenate([edge_attr, jnp.zeros((pad_e,), jnp.float32)])
    ew2 = ew.reshape(E_PAD // CHUNK, CHUNK)

    x_pad = jnp.zeros((N_PAD, 2), jnp.float32).at[:N_REAL].set(x)
    g = jnp.zeros((N_PAD, CP), jnp.float32).at[:N_REAL, :2].set(x)

    h = x_pad
    n_layers = len(conv_Wrel)
    for l in range(n_layers):
        c_in = conv_Wrel[l].shape[1]
        c_out = conv_Wrel[l].shape[0]
        aggp = _sc_edge(g, src2, dst2, ew2)
        first = (l == 0)
        c_mid = c_in if first else c_out
        wnext_t = conv_Wrel[l + 1].T if l + 1 < n_layers else None
        res = _conv_layer_tc(first, c_mid, c_in, c_out,
                             None if wnext_t is None else wnext_t.shape[1],
                             aggp, h, conv_Wrel[l].T, conv_brel[l],
                             conv_Wroot[l].T, wnext_t)
        if wnext_t is None:
            h = res
        else:
            h, g = res

    h5 = h[:N_REAL].reshape(-1, 396)
    rows = h5.shape[0]
    rows_pad = ((rows + 7) // 8) * 8
    h5p = jnp.zeros((rows_pad, 396), jnp.float32).at[:rows].set(h5)
    out = _mlp_tc(h5p, lin_W, lin_b)
    return out[:rows]
